# baseline (speedup 1.0000x reference)
import jax
import jax.numpy as jnp
from jax.experimental import pallas as pl

_BM1 = 200
_BM2 = 400


def _act0(s, b0, a0):
    o = s + b0
    return jnp.maximum(jnp.where(o >= 0.0, o, a0 * o), 0.0)


def _act1(u, b1, a1):
    o = u + b1
    return jnp.where(o >= 0.0, o, a1 * o)


def _bdot(a, b):
    return jnp.dot(a.astype(jnp.bfloat16), b, preferred_element_type=jnp.float32)


def _proj_kernel(x1_ref, x2_ref, w0t_ref, out_ref):
    w0t = w0t_ref[...]
    c = w0t.shape[1]
    p1 = jnp.dot(x1_ref[...], w0t, preferred_element_type=jnp.float32)
    p2 = jnp.dot(x2_ref[...], w0t, preferred_element_type=jnp.float32)
    out_ref[:, :c] = p1.astype(jnp.bfloat16)
    out_ref[:, c:] = p2.astype(jnp.bfloat16)


def _l1_all_kernel(nsteps, adj1_ref, adj2_ref, drop_ref, f_ref, b0_ref,
                   a0_ref, w1t_ref, t12_ref, t3_ref, t4_ref):
    i = pl.program_id(0)
    b0 = b0_ref[...]
    a0 = a0_ref[0, 0]
    w1t = w1t_ref[...]
    c = b0.shape[1]
    h = w1t.shape[1]

    @pl.when(i < nsteps)
    def _phase_adj1():
        s = _bdot(adj1_ref[...], f_ref[...])
        m1 = _act0(s[:, :c], b0, a0)
        m2 = _act0(s[:, c:], b0, a0)
        t12_ref[:, :h] = _bdot(m1, w1t).astype(jnp.bfloat16)
        t12_ref[:, h:] = _bdot(m2, w1t).astype(jnp.bfloat16)

    @pl.when((i >= nsteps) & (i < 2 * nsteps))
    def _phase_adj2():
        s = _bdot(adj2_ref[...], f_ref[:, :c])
        m = _act0(s, b0, a0)
        t3_ref[...] = _bdot(m, w1t).astype(jnp.bfloat16)

    @pl.when(i >= 2 * nsteps)
    def _phase_drop():
        s = _bdot(drop_ref[...], f_ref[:, :c])
        m = _act0(s, b0, a0)
        t4_ref[...] = _bdot(m, w1t).astype(jnp.bfloat16)


def _l2_pair_kernel(adj_ref, t_ref, b1_ref, a1_ref, h1_ref, h2_ref):
    u = _bdot(adj_ref[...], t_ref[...])
    b1 = b1_ref[...]
    a1 = a1_ref[0, 0]
    h = b1.shape[1]
    h1_ref[...] = _act1(u[:, :h], b1, a1)
    h2_ref[...] = _act1(u[:, h:], b1, a1)


def _l2_tail_kernel(nsteps, drop_ref, adj2_ref, t3_ref, t4_ref, b1_ref,
                    a1_ref, h1_ref, deg_ref, g1wt_ref, g1b_ref, g2wt_ref,
                    g2b_ref, gv1_ref, gv2_ref, gk_ref, h3_ref, h4_ref,
                    beta_ref):
    i = pl.program_id(0)
    b1 = b1_ref[...]
    a1 = a1_ref[0, 0]

    @pl.when(i < nsteps)
    def _phase_drop():
        u = _bdot(drop_ref[...], t4_ref[...])
        h4_ref[...] = _act1(u, b1, a1)

    @pl.when(i >= nsteps)
    def _phase_adj2():
        u = _bdot(adj2_ref[...], t3_ref[...])
        h3 = _act1(u, b1, a1)
        h3_ref[...] = h3
        z1 = jnp.dot(h1_ref[...], g1wt_ref[...],
                     preferred_element_type=jnp.float32) + g1b_ref[...]
        z2 = jnp.dot(h3, g2wt_ref[...],
                     preferred_element_type=jnp.float32) + g2b_ref[...]
        logit = (jnp.dot(z1, gv1_ref[...], preferred_element_type=jnp.float32)
                 + jnp.dot(z2, gv2_ref[...], preferred_element_type=jnp.float32)
                 + deg_ref[...] * gk_ref[0, 0] + gk_ref[0, 1])
        beta_ref[...] = jax.nn.sigmoid(logit)


def _full(shape):
    nd = len(shape)
    return pl.BlockSpec(shape, lambda i: (0,) * nd)


def _rows(bm, cols):
    return pl.BlockSpec((bm, cols), lambda i: (i, 0))


def _rows_ph(bm, cols, off, nb):
    return pl.BlockSpec(
        (bm, cols), lambda i: (jnp.clip(i - off, 0, nb - 1), 0))


def kernel(x1, x2, adj1, adj2, drop_edge_index, deg, W0, b0, a0, W1, b1, a1,
           g1W, g1b, g2W, g2b, g3W, g3b):
    n, d = x1.shape
    twoh = W0.shape[0]
    h = W1.shape[0]
    g = g1W.shape[0]
    f32 = jnp.float32
    bf16 = jnp.bfloat16

    w0t = W0.T
    w1t = W1.T.astype(bf16)
    b0r = b0.reshape(1, twoh)
    b1r = b1.reshape(1, h)
    a0r = a0.reshape(1, 1)
    a1r = a1.reshape(1, 1)
    g1wt = g1W.T
    g2wt = g2W.T
    g1br = g1b.reshape(1, g)
    g2br = g2b.reshape(1, g)
    gv1 = g3W[:, :g].T
    gv2 = g3W[:, g:2 * g].T
    gk = jnp.stack([g3W[0, 2 * g], g3b[0]]).reshape(1, 2)

    f0 = pl.pallas_call(
        _proj_kernel,
        grid=(pl.cdiv(n, _BM2),),
        in_specs=[_rows(_BM2, d), _rows(_BM2, d), _full((d, twoh))],
        out_specs=_rows(_BM2, 2 * twoh),
        out_shape=jax.ShapeDtypeStruct((n, 2 * twoh), bf16),
    )(x1, x2, w0t)

    ns1 = pl.cdiv(n, _BM1)
    t12, t3, t4 = pl.pallas_call(
        lambda *refs: _l1_all_kernel(ns1, *refs),
        grid=(3 * ns1,),
        in_specs=[_rows_ph(_BM1, n, 0, ns1), _rows_ph(_BM1, n, ns1, ns1),
                  _rows_ph(_BM1, n, 2 * ns1, ns1), _full((n, 2 * twoh)),
                  _full((1, twoh)), _full((1, 1)), _full((twoh, h))],
        out_specs=(_rows_ph(_BM1, 2 * h, 0, ns1),
                   _rows_ph(_BM1, h, ns1, ns1),
                   _rows_ph(_BM1, h, 2 * ns1, ns1)),
        out_shape=(jax.ShapeDtypeStruct((n, 2 * h), bf16),
                   jax.ShapeDtypeStruct((n, h), bf16),
                   jax.ShapeDtypeStruct((n, h), bf16)),
    )(adj1, adj2, drop_edge_index, f0, b0r, a0r, w1t)

    h1, h2 = pl.pallas_call(
        _l2_pair_kernel,
        grid=(pl.cdiv(n, _BM2),),
        in_specs=[_rows(_BM2, n), _full((n, 2 * h)), _full((1, h)),
                  _full((1, 1))],
        out_specs=(_rows(_BM2, h), _rows(_BM2, h)),
        out_shape=(jax.ShapeDtypeStruct((n, h), f32),
                   jax.ShapeDtypeStruct((n, h), f32)),
    )(adj1, t12, b1r, a1r)

    ns2 = pl.cdiv(n, _BM1)
    h3, h4, beta = pl.pallas_call(
        lambda *refs: _l2_tail_kernel(ns2, *refs),
        grid=(2 * ns2,),
        in_specs=[_rows_ph(_BM1, n, 0, ns2), _rows_ph(_BM1, n, ns2, ns2),
                  _full((n, h)), _full((n, h)), _full((1, h)), _full((1, 1)),
                  _rows_ph(_BM1, h, ns2, ns2), _rows_ph(_BM1, 1, ns2, ns2),
                  _full((h, g)), _full((1, g)), _full((h, g)), _full((1, g)),
                  _full((g, 1)), _full((g, 1)), _full((1, 2))],
        out_specs=(_rows_ph(_BM1, h, ns2, ns2), _rows_ph(_BM1, h, 0, ns2),
                   _rows_ph(_BM1, 1, ns2, ns2)),
        out_shape=(jax.ShapeDtypeStruct((n, h), f32),
                   jax.ShapeDtypeStruct((n, h), f32),
                   jax.ShapeDtypeStruct((n, 1), f32)),
    )(drop_edge_index, adj2, t3, t4, b1r, a1r, h1, deg,
      g1wt, g1br, g2wt, g2br, gv1, gv2, gk)

    return (h1, h2, h3, h4, beta)


# speedup vs baseline: 1.0218x; 1.0218x over previous
import jax
import jax.numpy as jnp
from jax.experimental import pallas as pl
from jax.experimental.pallas import tpu as pltpu

_BM = 400


def _act0(s, b0, a0):
    o = s + b0
    return jnp.maximum(jnp.where(o >= 0.0, o, a0 * o), 0.0)


def _act1(u, b1, a1):
    o = u + b1
    return jnp.where(o >= 0.0, o, a1 * o)


def _proj_kernel(x1_ref, x2_ref, w0t_ref, out_ref):
    w0t = w0t_ref[...]
    c = w0t.shape[1]
    p1 = jnp.dot(x1_ref[...], w0t, preferred_element_type=jnp.float32)
    p2 = jnp.dot(x2_ref[...], w0t, preferred_element_type=jnp.float32)
    out_ref[:, :c] = p1.astype(jnp.bfloat16)
    out_ref[:, c:] = p2.astype(jnp.bfloat16)


def _bdot(a, b):
    return jnp.dot(a.astype(jnp.bfloat16), b, preferred_element_type=jnp.float32)


def _l1_pair_kernel(adj_ref, f_ref, b0_ref, a0_ref, w1t_ref, out_ref):
    s = _bdot(adj_ref[...], f_ref[...])
    b0 = b0_ref[...]
    a0 = a0_ref[0, 0]
    w1t = w1t_ref[...]
    c = b0.shape[1]
    h = w1t.shape[1]
    m1 = _act0(s[:, :c], b0, a0)
    m2 = _act0(s[:, c:], b0, a0)
    out_ref[:, :h] = _bdot(m1, w1t).astype(jnp.bfloat16)
    out_ref[:, h:] = _bdot(m2, w1t).astype(jnp.bfloat16)


def _l1_one_kernel(adj_ref, f_ref, b0_ref, a0_ref, w1t_ref, out_ref):
    s = _bdot(adj_ref[...], f_ref[...])
    m = _act0(s, b0_ref[...], a0_ref[0, 0])
    out_ref[...] = _bdot(m, w1t_ref[...]).astype(jnp.bfloat16)


def _l2_pair_kernel(adj_ref, t_ref, b1_ref, a1_ref, h1_ref, h2_ref):
    u = _bdot(adj_ref[...], t_ref[...])
    b1 = b1_ref[...]
    a1 = a1_ref[0, 0]
    h = b1.shape[1]
    h1_ref[...] = _act1(u[:, :h], b1, a1)
    h2_ref[...] = _act1(u[:, h:], b1, a1)


def _l2_one_kernel(adj_ref, t_ref, b1_ref, a1_ref, h_ref):
    u = _bdot(adj_ref[...], t_ref[...])
    h_ref[...] = _act1(u, b1_ref[...], a1_ref[0, 0])


def _l2_beta_kernel(adj_ref, t_ref, b1_ref, a1_ref, h1_ref, deg_ref,
                    g1wt_ref, g1b_ref, g2wt_ref, g2b_ref,
                    gv1_ref, gv2_ref, gk_ref, h3_ref, beta_ref):
    u = _bdot(adj_ref[...], t_ref[...])
    h3 = _act1(u, b1_ref[...], a1_ref[0, 0])
    h3_ref[...] = h3
    z1 = jnp.dot(h1_ref[...], g1wt_ref[...], preferred_element_type=jnp.float32) + g1b_ref[...]
    z2 = jnp.dot(h3, g2wt_ref[...], preferred_element_type=jnp.float32) + g2b_ref[...]
    logit = (jnp.dot(z1, gv1_ref[...], preferred_element_type=jnp.float32)
             + jnp.dot(z2, gv2_ref[...], preferred_element_type=jnp.float32)
             + deg_ref[...] * gk_ref[0, 0] + gk_ref[0, 1])
    beta_ref[...] = jax.nn.sigmoid(logit)


def _full(shape):
    nd = len(shape)
    return pl.BlockSpec(shape, lambda i: (0,) * nd)


def _rows(bm, cols):
    return pl.BlockSpec((bm, cols), lambda i: (i, 0))


def kernel(x1, x2, adj1, adj2, drop_edge_index, deg, W0, b0, a0, W1, b1, a1,
           g1W, g1b, g2W, g2b, g3W, g3b):
    n, d = x1.shape
    twoh = W0.shape[0]
    h = W1.shape[0]
    g = g1W.shape[0]
    f32 = jnp.float32
    grid = (pl.cdiv(n, _BM),)

    bf16 = jnp.bfloat16
    w0t = W0.T
    w1t = W1.T.astype(bf16)
    b0r = b0.reshape(1, twoh)
    b1r = b1.reshape(1, h)
    a0r = a0.reshape(1, 1)
    a1r = a1.reshape(1, 1)
    g1wt = g1W.T
    g2wt = g2W.T
    g1br = g1b.reshape(1, g)
    g2br = g2b.reshape(1, g)
    gv1 = g3W[:, :g].T
    gv2 = g3W[:, g:2 * g].T
    gk = jnp.stack([g3W[0, 2 * g], g3b[0]]).reshape(1, 2)

    f0 = pl.pallas_call(
        _proj_kernel,
        grid=grid,
        in_specs=[_rows(_BM, d), _rows(_BM, d), _full((d, twoh))],
        out_specs=_rows(_BM, 2 * twoh),
        out_shape=jax.ShapeDtypeStruct((n, 2 * twoh), bf16),
        compiler_params=pltpu.CompilerParams(dimension_semantics=("parallel",)),
    )(x1, x2, w0t)
    f1 = f0[:, :twoh]

    t12 = pl.pallas_call(
        _l1_pair_kernel,
        grid=grid,
        in_specs=[_rows(_BM, n), _full((n, 2 * twoh)), _full((1, twoh)),
                  _full((1, 1)), _full((twoh, h))],
        out_specs=_rows(_BM, 2 * h),
        out_shape=jax.ShapeDtypeStruct((n, 2 * h), bf16),
        compiler_params=pltpu.CompilerParams(dimension_semantics=("parallel",)),
    )(adj1, f0, b0r, a0r, w1t)

    def l1_one(adj):
        return pl.pallas_call(
            _l1_one_kernel,
            grid=grid,
            in_specs=[_rows(_BM, n), _full((n, twoh)), _full((1, twoh)),
                      _full((1, 1)), _full((twoh, h))],
            out_specs=_rows(_BM, h),
            out_shape=jax.ShapeDtypeStruct((n, h), bf16),
            compiler_params=pltpu.CompilerParams(dimension_semantics=("parallel",)),
        )(adj, f1, b0r, a0r, w1t)

    t3 = l1_one(adj2)
    t4 = l1_one(drop_edge_index)

    h1, h2 = pl.pallas_call(
        _l2_pair_kernel,
        grid=grid,
        in_specs=[_rows(_BM, n), _full((n, 2 * h)), _full((1, h)),
                  _full((1, 1))],
        out_specs=(_rows(_BM, h), _rows(_BM, h)),
        out_shape=(jax.ShapeDtypeStruct((n, h), f32),
                   jax.ShapeDtypeStruct((n, h), f32)),
        compiler_params=pltpu.CompilerParams(dimension_semantics=("parallel",)),
    )(adj1, t12, b1r, a1r)

    h4 = pl.pallas_call(
        _l2_one_kernel,
        grid=grid,
        in_specs=[_rows(_BM, n), _full((n, h)), _full((1, h)), _full((1, 1))],
        out_specs=_rows(_BM, h),
        out_shape=jax.ShapeDtypeStruct((n, h), f32),
        compiler_params=pltpu.CompilerParams(dimension_semantics=("parallel",)),
    )(drop_edge_index, t4, b1r, a1r)

    h3, beta = pl.pallas_call(
        _l2_beta_kernel,
        grid=grid,
        in_specs=[_rows(_BM, n), _full((n, h)), _full((1, h)), _full((1, 1)),
                  _rows(_BM, h), _rows(_BM, 1), _full((h, g)), _full((1, g)),
                  _full((h, g)), _full((1, g)), _full((g, 1)), _full((g, 1)),
                  _full((1, 2))],
        out_specs=(_rows(_BM, h), _rows(_BM, 1)),
        out_shape=(jax.ShapeDtypeStruct((n, h), f32),
                   jax.ShapeDtypeStruct((n, 1), f32)),
        compiler_params=pltpu.CompilerParams(dimension_semantics=("parallel",)),
    )(adj2, t3, b1r, a1r, h1, deg, g1wt, g1br, g2wt, g2br, gv1, gv2, gk)

    return (h1, h2, h3, h4, beta)


# speedup vs baseline: 1.0440x; 1.0217x over previous
import jax
import jax.numpy as jnp
from jax.experimental import pallas as pl
from jax.experimental.pallas import tpu as pltpu

_BM = 400


def _act0(s, b0, a0):
    o = s + b0
    return jnp.maximum(jnp.where(o >= 0.0, o, a0 * o), 0.0)


def _act1(u, b1, a1):
    o = u + b1
    return jnp.where(o >= 0.0, o, a1 * o)


def _bdot(a, b):
    return jnp.dot(a.astype(jnp.bfloat16), b, preferred_element_type=jnp.float32)


def _l1_pair_kernel(adj_ref, x1_ref, x2_ref, w0t_ref, b0_ref, a0_ref,
                    w1t_ref, out_ref, f_ref):
    c = w0t_ref.shape[1]

    @pl.when(pl.program_id(0) == 0)
    def _proj():
        w0t = w0t_ref[...]
        p1 = jnp.dot(x1_ref[...], w0t, preferred_element_type=jnp.float32)
        p2 = jnp.dot(x2_ref[...], w0t, preferred_element_type=jnp.float32)
        f_ref[:, :c] = p1.astype(jnp.bfloat16)
        f_ref[:, c:] = p2.astype(jnp.bfloat16)

    s = _bdot(adj_ref[...], f_ref[...])
    b0 = b0_ref[...]
    a0 = a0_ref[0, 0]
    w1t = w1t_ref[...]
    h = w1t.shape[1]
    m1 = _act0(s[:, :c], b0, a0)
    m2 = _act0(s[:, c:], b0, a0)
    out_ref[:, :h] = _bdot(m1, w1t).astype(jnp.bfloat16)
    out_ref[:, h:] = _bdot(m2, w1t).astype(jnp.bfloat16)


def _l1_one_kernel(adj_ref, x1_ref, w0t_ref, b0_ref, a0_ref, w1t_ref,
                   out_ref, f_ref):
    @pl.when(pl.program_id(0) == 0)
    def _proj():
        p1 = jnp.dot(x1_ref[...], w0t_ref[...],
                     preferred_element_type=jnp.float32)
        f_ref[...] = p1.astype(jnp.bfloat16)

    s = _bdot(adj_ref[...], f_ref[...])
    m = _act0(s, b0_ref[...], a0_ref[0, 0])
    out_ref[...] = _bdot(m, w1t_ref[...]).astype(jnp.bfloat16)


def _l2_pair_kernel(adj_ref, t_ref, b1_ref, a1_ref, h1_ref, h2_ref):
    u = _bdot(adj_ref[...], t_ref[...])
    b1 = b1_ref[...]
    a1 = a1_ref[0, 0]
    h = b1.shape[1]
    h1_ref[...] = _act1(u[:, :h], b1, a1)
    h2_ref[...] = _act1(u[:, h:], b1, a1)


def _l2_one_kernel(adj_ref, t_ref, b1_ref, a1_ref, h_ref):
    u = _bdot(adj_ref[...], t_ref[...])
    h_ref[...] = _act1(u, b1_ref[...], a1_ref[0, 0])


def _l2_beta_kernel(adj_ref, t_ref, b1_ref, a1_ref, h1_ref, deg_ref,
                    g1wt_ref, g1b_ref, g2wt_ref, g2b_ref,
                    gv1_ref, gv2_ref, gk_ref, h3_ref, beta_ref):
    u = _bdot(adj_ref[...], t_ref[...])
    h3 = _act1(u, b1_ref[...], a1_ref[0, 0])
    h3_ref[...] = h3
    z1 = jnp.dot(h1_ref[...], g1wt_ref[...], preferred_element_type=jnp.float32) + g1b_ref[...]
    z2 = jnp.dot(h3, g2wt_ref[...], preferred_element_type=jnp.float32) + g2b_ref[...]
    logit = (jnp.dot(z1, gv1_ref[...], preferred_element_type=jnp.float32)
             + jnp.dot(z2, gv2_ref[...], preferred_element_type=jnp.float32)
             + deg_ref[...] * gk_ref[0, 0] + gk_ref[0, 1])
    beta_ref[...] = jax.nn.sigmoid(logit)


def _full(shape):
    nd = len(shape)
    return pl.BlockSpec(shape, lambda i: (0,) * nd)


def _rows(bm, cols):
    return pl.BlockSpec((bm, cols), lambda i: (i, 0))


def kernel(x1, x2, adj1, adj2, drop_edge_index, deg, W0, b0, a0, W1, b1, a1,
           g1W, g1b, g2W, g2b, g3W, g3b):
    n, d = x1.shape
    twoh = W0.shape[0]
    h = W1.shape[0]
    g = g1W.shape[0]
    f32 = jnp.float32
    grid = (pl.cdiv(n, _BM),)

    bf16 = jnp.bfloat16
    w0t = W0.T
    w1t = W1.T.astype(bf16)
    b0r = b0.reshape(1, twoh)
    b1r = b1.reshape(1, h)
    a0r = a0.reshape(1, 1)
    a1r = a1.reshape(1, 1)
    g1wt = g1W.T
    g2wt = g2W.T
    g1br = g1b.reshape(1, g)
    g2br = g2b.reshape(1, g)
    gv1 = g3W[:, :g].T
    gv2 = g3W[:, g:2 * g].T
    gk = jnp.stack([g3W[0, 2 * g], g3b[0]]).reshape(1, 2)

    t12 = pl.pallas_call(
        _l1_pair_kernel,
        grid=grid,
        in_specs=[_rows(_BM, n), _full((n, d)), _full((n, d)),
                  _full((d, twoh)), _full((1, twoh)), _full((1, 1)),
                  _full((twoh, h))],
        out_specs=_rows(_BM, 2 * h),
        out_shape=jax.ShapeDtypeStruct((n, 2 * h), bf16),
        scratch_shapes=[pltpu.VMEM((n, 2 * twoh), bf16)],
        compiler_params=pltpu.CompilerParams(dimension_semantics=("arbitrary",)),
    )(adj1, x1, x2, w0t, b0r, a0r, w1t)

    def l1_one(adj):
        return pl.pallas_call(
            _l1_one_kernel,
            grid=grid,
            in_specs=[_rows(_BM, n), _full((n, d)), _full((d, twoh)),
                      _full((1, twoh)), _full((1, 1)), _full((twoh, h))],
            out_specs=_rows(_BM, h),
            out_shape=jax.ShapeDtypeStruct((n, h), bf16),
            scratch_shapes=[pltpu.VMEM((n, twoh), bf16)],
            compiler_params=pltpu.CompilerParams(dimension_semantics=("arbitrary",)),
        )(adj, x1, w0t, b0r, a0r, w1t)

    t3 = l1_one(adj2)
    t4 = l1_one(drop_edge_index)

    h1, h2 = pl.pallas_call(
        _l2_pair_kernel,
        grid=grid,
        in_specs=[_rows(_BM, n), _full((n, 2 * h)), _full((1, h)),
                  _full((1, 1))],
        out_specs=(_rows(_BM, h), _rows(_BM, h)),
        out_shape=(jax.ShapeDtypeStruct((n, h), f32),
                   jax.ShapeDtypeStruct((n, h), f32)),
        compiler_params=pltpu.CompilerParams(dimension_semantics=("parallel",)),
    )(adj1, t12, b1r, a1r)

    h4 = pl.pallas_call(
        _l2_one_kernel,
        grid=grid,
        in_specs=[_rows(_BM, n), _full((n, h)), _full((1, h)), _full((1, 1))],
        out_specs=_rows(_BM, h),
        out_shape=jax.ShapeDtypeStruct((n, h), f32),
        compiler_params=pltpu.CompilerParams(dimension_semantics=("parallel",)),
    )(drop_edge_index, t4, b1r, a1r)

    h3, beta = pl.pallas_call(
        _l2_beta_kernel,
        grid=grid,
        in_specs=[_rows(_BM, n), _full((n, h)), _full((1, h)), _full((1, 1)),
                  _rows(_BM, h), _rows(_BM, 1), _full((h, g)), _full((1, g)),
                  _full((h, g)), _full((1, g)), _full((g, 1)), _full((g, 1)),
                  _full((1, 2))],
        out_specs=(_rows(_BM, h), _rows(_BM, 1)),
        out_shape=(jax.ShapeDtypeStruct((n, h), f32),
                   jax.ShapeDtypeStruct((n, 1), f32)),
        compiler_params=pltpu.CompilerParams(dimension_semantics=("parallel",)),
    )(adj2, t3, b1r, a1r, h1, deg, g1wt, g1br, g2wt, g2br, gv1, gv2, gk)

    return (h1, h2, h3, h4, beta)


# speedup vs baseline: 1.0581x; 1.0135x over previous
import jax
import jax.numpy as jnp
from jax.experimental import pallas as pl
from jax.experimental.pallas import tpu as pltpu

_BM = 400
_BM2 = 280


def _act0(s, b0, a0):
    o = s + b0
    return jnp.maximum(jnp.where(o >= 0.0, o, a0 * o), 0.0)


def _act1(u, b1, a1):
    o = u + b1
    return jnp.where(o >= 0.0, o, a1 * o)


def _bdot(a, b):
    return jnp.dot(a.astype(jnp.bfloat16), b, preferred_element_type=jnp.float32)


def _l1_pair_kernel(adj_ref, x1_ref, x2_ref, w0t_ref, b0_ref, a0_ref,
                    w1t_ref, out_ref, f_ref):
    c = w0t_ref.shape[1]

    @pl.when(pl.program_id(0) == 0)
    def _proj():
        w0t = w0t_ref[...]
        p1 = jnp.dot(x1_ref[...], w0t, preferred_element_type=jnp.float32)
        p2 = jnp.dot(x2_ref[...], w0t, preferred_element_type=jnp.float32)
        f_ref[:, :c] = p1.astype(jnp.bfloat16)
        f_ref[:, c:] = p2.astype(jnp.bfloat16)

    s = _bdot(adj_ref[...], f_ref[...])
    b0 = b0_ref[...]
    a0 = a0_ref[0, 0]
    w1t = w1t_ref[...]
    h = w1t.shape[1]
    m1 = _act0(s[:, :c], b0, a0)
    m2 = _act0(s[:, c:], b0, a0)
    out_ref[:, :h] = _bdot(m1, w1t).astype(jnp.bfloat16)
    out_ref[:, h:] = _bdot(m2, w1t).astype(jnp.bfloat16)


def _l1_two_kernel(ns, adj2_ref, drop_ref, x1_ref, w0t_ref, b0_ref, a0_ref,
                   w1t_ref, t3_ref, t4_ref, f_ref):
    i = pl.program_id(0)

    @pl.when(i == 0)
    def _proj():
        p1 = jnp.dot(x1_ref[...], w0t_ref[...],
                     preferred_element_type=jnp.float32)
        f_ref[...] = p1.astype(jnp.bfloat16)

    b0 = b0_ref[...]
    a0 = a0_ref[0, 0]
    w1t = w1t_ref[...]

    @pl.when(i < ns)
    def _adj2():
        s = _bdot(adj2_ref[...], f_ref[...])
        m = _act0(s, b0, a0)
        t3_ref[...] = _bdot(m, w1t).astype(jnp.bfloat16)

    @pl.when(i >= ns)
    def _drop():
        s = _bdot(drop_ref[...], f_ref[...])
        m = _act0(s, b0, a0)
        t4_ref[...] = _bdot(m, w1t).astype(jnp.bfloat16)


def _l2_pair_kernel(adj_ref, t_ref, b1_ref, a1_ref, h1_ref, h2_ref):
    u = _bdot(adj_ref[...], t_ref[...])
    b1 = b1_ref[...]
    a1 = a1_ref[0, 0]
    h = b1.shape[1]
    h1_ref[...] = _act1(u[:, :h], b1, a1)
    h2_ref[...] = _act1(u[:, h:], b1, a1)


def _l2_two_kernel(ns, drop_ref, adj2_ref, t4_ref, t3_ref, b1_ref, a1_ref,
                   h1_ref, deg_ref, g1wt_ref, g1b_ref, g2wt_ref, g2b_ref,
                   gv1_ref, gv2_ref, gk_ref, h4_ref, h3_ref, beta_ref):
    i = pl.program_id(0)
    b1 = b1_ref[...]
    a1 = a1_ref[0, 0]

    @pl.when(i < ns)
    def _drop():
        u = _bdot(drop_ref[...], t4_ref[...])
        h4_ref[...] = _act1(u, b1, a1)

    @pl.when(i >= ns)
    def _adj2():
        u = _bdot(adj2_ref[...], t3_ref[...])
        h3 = _act1(u, b1, a1)
        h3_ref[...] = h3
        z1 = jnp.dot(h1_ref[...], g1wt_ref[...],
                     preferred_element_type=jnp.float32) + g1b_ref[...]
        z2 = jnp.dot(h3, g2wt_ref[...],
                     preferred_element_type=jnp.float32) + g2b_ref[...]
        logit = (jnp.dot(z1, gv1_ref[...], preferred_element_type=jnp.float32)
                 + jnp.dot(z2, gv2_ref[...], preferred_element_type=jnp.float32)
                 + deg_ref[...] * gk_ref[0, 0] + gk_ref[0, 1])
        beta_ref[...] = jax.nn.sigmoid(logit)


def _full(shape):
    nd = len(shape)
    return pl.BlockSpec(shape, lambda i: (0,) * nd)


def _rows(bm, cols):
    return pl.BlockSpec((bm, cols), lambda i: (i, 0))


def _rows_ph(bm, cols, off, nb):
    return pl.BlockSpec(
        (bm, cols), lambda i: (jnp.clip(i - off, 0, nb - 1), 0))


def kernel(x1, x2, adj1, adj2, drop_edge_index, deg, W0, b0, a0, W1, b1, a1,
           g1W, g1b, g2W, g2b, g3W, g3b):
    n, d = x1.shape
    twoh = W0.shape[0]
    h = W1.shape[0]
    g = g1W.shape[0]
    f32 = jnp.float32
    grid = (pl.cdiv(n, _BM),)

    bf16 = jnp.bfloat16
    w0t = W0.T
    w1t = W1.T.astype(bf16)
    b0r = b0.reshape(1, twoh)
    b1r = b1.reshape(1, h)
    a0r = a0.reshape(1, 1)
    a1r = a1.reshape(1, 1)
    g1wt = g1W.T
    g2wt = g2W.T
    g1br = g1b.reshape(1, g)
    g2br = g2b.reshape(1, g)
    gv1 = g3W[:, :g].T
    gv2 = g3W[:, g:2 * g].T
    gk = jnp.stack([g3W[0, 2 * g], g3b[0]]).reshape(1, 2)

    t12 = pl.pallas_call(
        _l1_pair_kernel,
        grid=grid,
        in_specs=[_rows(_BM, n), _full((n, d)), _full((n, d)),
                  _full((d, twoh)), _full((1, twoh)), _full((1, 1)),
                  _full((twoh, h))],
        out_specs=_rows(_BM, 2 * h),
        out_shape=jax.ShapeDtypeStruct((n, 2 * h), bf16),
        scratch_shapes=[pltpu.VMEM((n, 2 * twoh), bf16)],
        compiler_params=pltpu.CompilerParams(dimension_semantics=("arbitrary",)),
    )(adj1, x1, x2, w0t, b0r, a0r, w1t)

    ns = pl.cdiv(n, _BM2)
    t3, t4 = pl.pallas_call(
        lambda *refs: _l1_two_kernel(ns, *refs),
        grid=(2 * ns,),
        in_specs=[_rows_ph(_BM2, n, 0, ns), _rows_ph(_BM2, n, ns, ns),
                  _full((n, d)), _full((d, twoh)), _full((1, twoh)),
                  _full((1, 1)), _full((twoh, h))],
        out_specs=(_rows_ph(_BM2, h, 0, ns), _rows_ph(_BM2, h, ns, ns)),
        out_shape=(jax.ShapeDtypeStruct((n, h), bf16),
                   jax.ShapeDtypeStruct((n, h), bf16)),
        scratch_shapes=[pltpu.VMEM((n, twoh), bf16)],
        compiler_params=pltpu.CompilerParams(dimension_semantics=("arbitrary",)),
    )(adj2, drop_edge_index, x1, w0t, b0r, a0r, w1t)

    h1, h2 = pl.pallas_call(
        _l2_pair_kernel,
        grid=grid,
        in_specs=[_rows(_BM, n), _full((n, 2 * h)), _full((1, h)),
                  _full((1, 1))],
        out_specs=(_rows(_BM, h), _rows(_BM, h)),
        out_shape=(jax.ShapeDtypeStruct((n, h), f32),
                   jax.ShapeDtypeStruct((n, h), f32)),
        compiler_params=pltpu.CompilerParams(dimension_semantics=("parallel",)),
    )(adj1, t12, b1r, a1r)

    h4, h3, beta = pl.pallas_call(
        lambda *refs: _l2_two_kernel(ns, *refs),
        grid=(2 * ns,),
        in_specs=[_rows_ph(_BM2, n, 0, ns), _rows_ph(_BM2, n, ns, ns),
                  _full((n, h)), _full((n, h)), _full((1, h)), _full((1, 1)),
                  _rows_ph(_BM2, h, ns, ns), _rows_ph(_BM2, 1, ns, ns),
                  _full((h, g)), _full((1, g)), _full((h, g)), _full((1, g)),
                  _full((g, 1)), _full((g, 1)), _full((1, 2))],
        out_specs=(_rows_ph(_BM2, h, 0, ns), _rows_ph(_BM2, h, ns, ns),
                   _rows_ph(_BM2, 1, ns, ns)),
        out_shape=(jax.ShapeDtypeStruct((n, h), f32),
                   jax.ShapeDtypeStruct((n, h), f32),
                   jax.ShapeDtypeStruct((n, 1), f32)),
        compiler_params=pltpu.CompilerParams(dimension_semantics=("arbitrary",)),
    )(drop_edge_index, adj2, t4, t3, b1r, a1r, h1, deg,
      g1wt, g1br, g2wt, g2br, gv1, gv2, gk)

    return (h1, h2, h3, h4, beta)
